# baseline (speedup 1.0000x reference)
import functools

import jax
import jax.numpy as jnp
from jax import lax
from jax.experimental import pallas as pl
from jax.experimental.pallas import tpu as pltpu
from jax.experimental.pallas import tpu_sc as plsc

U = 100000; I = 100000; D = 64; B = 4096; L = 30; NB = 4; NL = 2
NC = 2
NS = 16
LANES = 16
NQ = 4
NOUT = 100000
ACC_ROWS = 100096
MACRO = 12
EPM = MACRO * 128
CHUNK = NS * EPM


def _mesh():
    return plsc.VectorSubcoreMesh(
        core_axis_name="c", subcore_axis_name="s",
        num_cores=NC, num_subcores=NS)


def _make_spmm(n_macro):
    zper = ACC_ROWS // NS
    zchunks = []
    off = 0
    while off < zper:
        sz = min(EPM, zper - off)
        zchunks.append((off, sz))
        off += sz

    @functools.partial(
        pl.kernel,
        out_type=jax.ShapeDtypeStruct((NQ, ACC_ROWS, LANES), jnp.float32),
        mesh=_mesh(),
        scratch_types=[
            pltpu.VMEM((MACRO, 3, 128), jnp.int32),
            pltpu.VMEM((EPM, LANES), jnp.float32),
            pltpu.VMEM_SHARED((ACC_ROWS, LANES), jnp.float32),
            pltpu.SemaphoreType.DMA,
            pltpu.SemaphoreType.DMA,
        ],
        compiler_params=pltpu.CompilerParams(use_tc_tiling_on_sc=False),
    )
    def spmm(epk, x4, out, ebuf, gbuf, acc, gsem, ssem):
        c = lax.axis_index("c")
        s = lax.axis_index("s")

        for q in range(2):
            qq = c * 2 + q

            @plsc.parallel_loop(0, EPM, unroll=4)
            def _zero(i):
                gbuf[i, :] = jnp.zeros((LANES,), jnp.float32)

            for (zoff, zsz) in zchunks:
                pltpu.sync_copy(gbuf.at[pl.ds(0, zsz)],
                                acc.at[pl.ds(s * zper + zoff, zsz)])
            plsc.subcore_barrier()

            def macro_body(m, _):
                base = (s * n_macro + m) * MACRO
                pltpu.sync_copy(epk.at[pl.ds(base, MACRO)], ebuf)
                for j in range(MACRO):
                    for v in range(8):
                        sl = pl.ds(v * LANES, LANES)
                        ebuf[j, 1, sl] = ebuf[j, 1, sl] + qq
                gcps = [pltpu.async_copy(x4.at[ebuf.at[j, 1]],
                                         gbuf.at[pl.ds(j * 128, 128)], gsem)
                        for j in range(MACRO)]
                for cp in gcps:
                    cp.wait()
                for j in range(MACRO):
                    @plsc.parallel_loop(0, 8)
                    def _scale(g):
                        vv = lax.bitcast_convert_type(
                            ebuf[j, 2, pl.ds(g * LANES, LANES)], jnp.float32)
                        be_ = j * 128 + g * LANES
                        for i in range(LANES):
                            gbuf[be_ + i, :] = gbuf[be_ + i, :] * vv[i]
                scps = [pltpu.async_copy(gbuf.at[pl.ds(j * 128, 128)],
                                         acc.at[ebuf.at[j, 0]], ssem, add=True)
                        for j in range(MACRO)]
                for cp in scps:
                    cp.wait()
                return 0

            lax.fori_loop(0, n_macro, macro_body, 0)
            plsc.subcore_barrier()
            pltpu.sync_copy(acc.at[pl.ds(s * zper, zper)],
                            out.at[qq, pl.ds(s * zper, zper), :])
            plsc.subcore_barrier()

    return spmm


def _make_gather(total):
    per_w = total // (NC * NS * 128)

    @functools.partial(
        pl.kernel,
        out_type=jax.ShapeDtypeStruct((total, D), jnp.float32),
        mesh=_mesh(),
        scratch_types=[
            pltpu.VMEM((per_w, 128), jnp.int32),
            pltpu.VMEM((2, 128, D), jnp.float32),
            pltpu.SemaphoreType.DMA,
            pltpu.SemaphoreType.DMA,
        ],
        compiler_params=pltpu.CompilerParams(use_tc_tiling_on_sc=False),
    )
    def gat(idx3d, x2, out, ibuf, gbuf, gsem, osem):
        c = lax.axis_index("c")
        s = lax.axis_index("s")
        w = s * NC + c
        pltpu.sync_copy(idx3d.at[w], ibuf)
        cps = [None] * per_w
        ocp = [None] * per_w
        for k in range(per_w):
            if k >= 2:
                ocp[k - 2].wait()
            cps[k] = pltpu.async_copy(x2.at[ibuf.at[k]], gbuf.at[k % 2], gsem)
            if k >= 1:
                cps[k - 1].wait()
                ocp[k - 1] = pltpu.async_copy(
                    gbuf.at[(k - 1) % 2],
                    out.at[pl.ds((w * per_w + k - 1) * 128, 128)], osem)
        kl = per_w - 1
        cps[kl].wait()
        ocp[kl] = pltpu.async_copy(
            gbuf.at[kl % 2], out.at[pl.ds((w * per_w + kl) * 128, 128)], osem)
        ocp[kl - 1].wait()
        ocp[kl].wait()

    return gat


def _transform_tc(x, a, w, bias):
    n = x.shape[0]
    bn = 2000

    def body(x_ref, a_ref, w_ref, b_ref, o_ref):
        t = x_ref[...] + a_ref[...]
        o_ref[...] = jnp.maximum(
            jnp.dot(t, w_ref[...], preferred_element_type=jnp.float32)
            + b_ref[...], 0.0)

    return pl.pallas_call(
        body,
        grid=(n // bn,),
        in_specs=[
            pl.BlockSpec((bn, D), lambda i: (i, 0)),
            pl.BlockSpec((bn, D), lambda i: (i, 0)),
            pl.BlockSpec((D, D), lambda i: (0, 0)),
            pl.BlockSpec((1, D), lambda i: (0, 0)),
        ],
        out_specs=pl.BlockSpec((bn, D), lambda i: (i, 0)),
        out_shape=jax.ShapeDtypeStruct((n, D), jnp.float32),
    )(x, a, w, bias)


BSEQ = 256
NBLK = B // BSEQ


def _ln_in(x, g, b):
    m = x.mean(-1, keepdims=True)
    v = ((x - m) ** 2).mean(-1, keepdims=True)
    return (x - m) / jnp.sqrt(v + 1e-5) * g + b


def _seq_body(se_ref, oh_ref, tx_ref, len_ref, ug_ref, pos_ref, neg_ref,
              pb_ref, tW1_ref, tb1_ref, tW2_ref, tb2_ref, wih_ref, whh_ref,
              bih_ref, bhh_ref, lng_ref, lnb_ref, out_ref, xscr):
    lng = lng_ref[...]
    lnb = lnb_ref[...]
    tx = tx_ref[...]
    h1 = jnp.maximum(tx * tW1_ref[...][None] + tb1_ref[...][None], 0.0)
    te = jnp.dot(h1.reshape(L * BSEQ, D), tW2_ref[...],
                 preferred_element_type=jnp.float32) + tb2_ref[...]
    be = jnp.dot(oh_ref[...].reshape(L * BSEQ, NB), wih_ref[...][3, :NB, :],
                 preferred_element_type=jnp.float32)
    x = se_ref[...].reshape(L * BSEQ, D) + be + te
    xscr[...] = _ln_in(x, lng, lnb).reshape(L, BSEQ, D)

    lens = len_ref[...]

    def step(t, carry):
        h, res = carry
        xt = xscr[t]
        gr = (jnp.dot(xt, wih_ref[...][0], preferred_element_type=jnp.float32)
              + jnp.dot(h, whh_ref[...][0], preferred_element_type=jnp.float32)
              + bih_ref[...][0] + bhh_ref[...][0])
        gz = (jnp.dot(xt, wih_ref[...][1], preferred_element_type=jnp.float32)
              + jnp.dot(h, whh_ref[...][1], preferred_element_type=jnp.float32)
              + bih_ref[...][1] + bhh_ref[...][1])
        r = jax.nn.sigmoid(gr)
        z = jax.nn.sigmoid(gz)
        hn = (jnp.dot(h, whh_ref[...][2], preferred_element_type=jnp.float32)
              + bhh_ref[...][2])
        inn = (jnp.dot(xt, wih_ref[...][2], preferred_element_type=jnp.float32)
               + bih_ref[...][2])
        n = jnp.tanh(inn + r * hn)
        hnew = (1.0 - z) * n + z * h
        res = jnp.where(lens == t + 1, hnew, res)
        return hnew, res

    h0 = jnp.zeros((BSEQ, D), jnp.float32)
    _, res = lax.fori_loop(0, L, step, (h0, h0))

    uf = _ln_in(ug_ref[...] + res, lng, lnb)
    pos = pos_ref[...]
    neg = neg_ref[...]
    ps = jnp.sum(uf * pos, axis=-1, keepdims=True)
    ns = jnp.sum(uf * neg, axis=-1, keepdims=True)
    xm = ps - ns
    sp = jnp.maximum(-xm, 0.0) + jnp.log1p(jnp.exp(-jnp.abs(xm)))
    pb = pb_ref[...]
    bw = jnp.where(pb == 0, 1.0,
                   jnp.where(pb == 1, 1.25, jnp.where(pb == 2, 1.6, 2.1)))
    out_ref[0, 0, 0] = jnp.sum(sp * bw)
    out_ref[0, 0, 1] = jnp.sum(jnp.sqrt(jnp.sum(uf * uf, axis=-1)))
    out_ref[0, 0, 2] = jnp.sum(jnp.sqrt(jnp.sum(pos * pos, axis=-1)))
    out_ref[0, 0, 3] = jnp.sum(jnp.sqrt(jnp.sum(neg * neg, axis=-1)))


def _seq_tc(seT, ohT, txT, lens, ug, pos, neg, pb, t_W1, t_b1, t_W2, t_b2,
            wih4, whh3, bih3, bhh3, ln_g, ln_b):
    return pl.pallas_call(
        _seq_body,
        grid=(NBLK,),
        in_specs=[
            pl.BlockSpec((L, BSEQ, D), lambda i: (0, i, 0)),
            pl.BlockSpec((L, BSEQ, NB), lambda i: (0, i, 0)),
            pl.BlockSpec((L, BSEQ, 1), lambda i: (0, i, 0)),
            pl.BlockSpec((BSEQ, 1), lambda i: (i, 0)),
            pl.BlockSpec((BSEQ, D), lambda i: (i, 0)),
            pl.BlockSpec((BSEQ, D), lambda i: (i, 0)),
            pl.BlockSpec((BSEQ, D), lambda i: (i, 0)),
            pl.BlockSpec((BSEQ, 1), lambda i: (i, 0)),
            pl.BlockSpec((1, D), lambda i: (0, 0)),
            pl.BlockSpec((1, D), lambda i: (0, 0)),
            pl.BlockSpec((D, D), lambda i: (0, 0)),
            pl.BlockSpec((1, D), lambda i: (0, 0)),
            pl.BlockSpec((4, D, D), lambda i: (0, 0, 0)),
            pl.BlockSpec((3, D, D), lambda i: (0, 0, 0)),
            pl.BlockSpec((3, 1, D), lambda i: (0, 0, 0)),
            pl.BlockSpec((3, 1, D), lambda i: (0, 0, 0)),
            pl.BlockSpec((1, D), lambda i: (0, 0)),
            pl.BlockSpec((1, D), lambda i: (0, 0)),
        ],
        out_specs=pl.BlockSpec((1, 1, 4), lambda i: (i, 0, 0),
                               memory_space=pltpu.SMEM),
        out_shape=jax.ShapeDtypeStruct((NBLK, 1, 4), jnp.float32),
        scratch_shapes=[pltpu.VMEM((L, BSEQ, D), jnp.float32)],
    )(seT, ohT, txT, lens, ug, pos, neg, pb, t_W1, t_b1, t_W2, t_b2,
      wih4, whh3, bih3, bhh3, ln_g, ln_b)


def _pad_edges(rows, cols, vals):
    n = rows.shape[0]
    npad = (-n) % CHUNK
    if npad:
        rows = jnp.concatenate(
            [rows, NOUT + (jnp.arange(npad, dtype=jnp.int32) % LANES)])
        cols = jnp.concatenate([cols, jnp.zeros((npad,), jnp.int32)])
        vals = jnp.concatenate([vals, jnp.zeros((npad,), jnp.float32)])
    total = n + npad
    n_macro = total // CHUNK
    epk = jnp.stack([
        rows.reshape(total // 128, 128),
        (cols * 4).reshape(total // 128, 128),
        lax.bitcast_convert_type(vals, jnp.int32).reshape(total // 128, 128),
    ], axis=1)
    return epk, n_macro


def kernel(ui_rows, ui_cols, ui_vals, ii_rows, ii_cols, ii_vals, seq_items, seq_behaviors, seq_delta_days, seq_len, user_idx, pos_item_idx, neg_item_idx, pos_behavior, user_emb, item_emb, beh_emb, gnn_u_W, gnn_u_b, gnn_i_W, gnn_i_b, t_W1, t_b1, t_W2, t_b2, gru_Wih, gru_Whh, gru_bih, gru_bhh, ln_g, ln_b):
    i32 = jnp.int32
    ui_rows = ui_rows.astype(i32)
    ui_cols = ui_cols.astype(i32)
    ii_rows = ii_rows.astype(i32)
    ii_cols = ii_cols.astype(i32)

    i_rows = jnp.concatenate([ii_rows, ui_cols])
    i_cols = jnp.concatenate([ii_cols, ui_rows + I])
    i_vals = jnp.concatenate([ii_vals, ui_vals])

    epk_u, nmu = _pad_edges(ui_rows, ui_cols, ui_vals)
    epk_i, nmi = _pad_edges(i_rows, i_cols, i_vals)
    spmm_u = _make_spmm(nmu)
    spmm_i = _make_spmm(nmi)

    u, it = user_emb, item_emb
    for l in range(NL):
        x4 = jnp.concatenate([it, u], axis=0).reshape((I + U) * NQ, LANES)
        agg_u4 = spmm_u(epk_u, x4)
        agg_i4 = spmm_i(epk_i, x4)
        agg_u = jnp.moveaxis(agg_u4, 0, 1).reshape(ACC_ROWS, D)[:NOUT]
        agg_i = jnp.moveaxis(agg_i4, 0, 1).reshape(ACC_ROWS, D)[:NOUT]
        u = _transform_tc(u, agg_u, gnn_u_W[l], gnn_u_b[l].reshape(1, D))
        it = _transform_tc(it, agg_i, gnn_i_W[l], gnn_i_b[l].reshape(1, D))

    x2 = jnp.concatenate([it, u], axis=0)
    gidx = jnp.concatenate([
        seq_items.reshape(-1).astype(i32),
        pos_item_idx.astype(i32),
        neg_item_idx.astype(i32),
        user_idx.astype(i32) + I,
    ])
    total = gidx.shape[0]
    g = _make_gather(total)(
        gidx.reshape(NC * NS, total // (NC * NS * 128), 128), x2)
    se = g[:B * L].reshape(B, L, D)
    pos = g[B * L:B * L + B]
    neg = g[B * L + B:B * L + 2 * B]
    ug = g[B * L + 2 * B:]

    seT = jnp.swapaxes(se, 0, 1)
    ohT = jnp.swapaxes(
        jax.nn.one_hot(seq_behaviors, NB, dtype=jnp.float32), 0, 1)
    txT = jnp.swapaxes(jnp.log1p(seq_delta_days), 0, 1)[..., None]
    wih3 = jnp.stack(jnp.split(gru_Wih, 3, axis=1))
    behp = jnp.zeros((1, D, D), jnp.float32).at[0, :NB, :].set(0.35 * beh_emb)
    wih4 = jnp.concatenate([wih3, behp], axis=0)
    whh3 = jnp.stack(jnp.split(gru_Whh, 3, axis=1))
    bih3 = jnp.stack(jnp.split(gru_bih, 3)).reshape(3, 1, D)
    bhh3 = jnp.stack(jnp.split(gru_bhh, 3)).reshape(3, 1, D)

    partials = _seq_tc(
        seT, ohT, txT, seq_len.astype(i32).reshape(B, 1), ug, pos, neg,
        pos_behavior.astype(i32).reshape(B, 1),
        t_W1, t_b1.reshape(1, D), t_W2, t_b2.reshape(1, D),
        wih4, whh3, bih3, bhh3, ln_g.reshape(1, D), ln_b.reshape(1, D))
    sums = partials.reshape(NBLK, 4).sum(0)
    bpr = sums[0] / B
    reg = (sums[1] + sums[2] + sums[3]) / B * 1e-4
    return bpr + reg


# speedup vs baseline: 2.6069x; 2.6069x over previous
import functools

import jax
import jax.numpy as jnp
from jax import lax
from jax.experimental import pallas as pl
from jax.experimental.pallas import tpu as pltpu
from jax.experimental.pallas import tpu_sc as plsc

U = 100000; I = 100000; D = 64; B = 4096; L = 30; NB = 4; NL = 2
NC = 2
NS = 16
LANES = 16
NQ = 4
NOUT = 100000
ACC_ROWS = 100096
MACRO = 4
EPM = MACRO * 128
NSLOT = 3
CHUNK = NS * EPM * NSLOT


def _mesh():
    return plsc.VectorSubcoreMesh(
        core_axis_name="c", subcore_axis_name="s",
        num_cores=NC, num_subcores=NS)


def _make_spmm(n_macro):
    zper = ACC_ROWS // NS
    zchunks = []
    off = 0
    while off < zper:
        sz = min(EPM, zper - off)
        zchunks.append((off, sz))
        off += sz
    n_tri = n_macro // NSLOT

    @functools.partial(
        pl.kernel,
        out_type=jax.ShapeDtypeStruct((NQ, ACC_ROWS, LANES), jnp.float32),
        mesh=_mesh(),
        scratch_types=[
            pltpu.VMEM((NSLOT, MACRO, 3, 128), jnp.int32),
            pltpu.VMEM((NSLOT, MACRO, 128), jnp.int32),
            pltpu.VMEM((NSLOT, EPM, LANES), jnp.float32),
            pltpu.VMEM_SHARED((ACC_ROWS, LANES), jnp.float32),
            pltpu.SemaphoreType.DMA,
            pltpu.SemaphoreType.DMA,
            pltpu.SemaphoreType.DMA,
        ],
        compiler_params=pltpu.CompilerParams(use_tc_tiling_on_sc=False),
    )
    def spmm(epk, x4, out, ebuf, sidx, gbuf, acc, gsem, ssem, esem):
        c = lax.axis_index("c")
        s = lax.axis_index("s")

        def drain_s(x):
            for j in range(MACRO):
                pltpu.make_async_copy(gbuf.at[x, pl.ds(j * 128, 128)],
                                      acc.at[sidx.at[x, j]], ssem).wait()

        def wait_e(x):
            pltpu.make_async_copy(epk.at[pl.ds(0, MACRO)], ebuf.at[x],
                                  esem).wait()

        def prefetch_e(x, m):
            base = (s * n_macro + m) * MACRO
            pltpu.async_copy(epk.at[pl.ds(base, MACRO)], ebuf.at[x], esem)

        def front(x, m, qq):
            drain_s(x)
            wait_e(x)
            for j in range(MACRO):
                for v in range(8):
                    sl = pl.ds(v * LANES, LANES)
                    ebuf[x, j, 1, sl] = ebuf[x, j, 1, sl] + qq
                    sidx[x, j, sl] = ebuf[x, j, 0, sl]
            return [pltpu.async_copy(x4.at[ebuf.at[x, j, 1]],
                                     gbuf.at[x, pl.ds(j * 128, 128)], gsem)
                    for j in range(MACRO)]

        def back(x, m, gcps):
            for cp in gcps:
                cp.wait()
            for j in range(MACRO):
                @plsc.parallel_loop(0, 8)
                def _scale(g):
                    vv = lax.bitcast_convert_type(
                        ebuf[x, j, 2, pl.ds(g * LANES, LANES)], jnp.float32)
                    be_ = j * 128 + g * LANES
                    for i in range(LANES):
                        gbuf[x, be_ + i, :] = gbuf[x, be_ + i, :] * vv[i]
            for j in range(MACRO):
                pltpu.async_copy(gbuf.at[x, pl.ds(j * 128, 128)],
                                 acc.at[sidx.at[x, j]], ssem, add=True)
            prefetch_e(x, jnp.minimum(m + NSLOT, n_macro - 1))

        def quarter(q, _):
            qq = c * 2 + q
            for x in range(NSLOT):
                @plsc.parallel_loop(0, EPM, unroll=4)
                def _zero(i):
                    gbuf[x, i, :] = jnp.zeros((LANES,), jnp.float32)
            for (zoff, zsz) in zchunks:
                pltpu.sync_copy(gbuf.at[0, pl.ds(0, zsz)],
                                acc.at[pl.ds(s * zper + zoff, zsz)])
            plsc.subcore_barrier()
            dump = NOUT + lax.iota(jnp.int32, LANES)
            for x in range(NSLOT):
                for j in range(MACRO):
                    for v in range(8):
                        sidx[x, j, pl.ds(v * LANES, LANES)] = dump
            for x in range(NSLOT):
                for j in range(MACRO):
                    pltpu.async_copy(gbuf.at[x, pl.ds(j * 128, 128)],
                                     acc.at[sidx.at[x, j]], ssem, add=True)
                prefetch_e(x, x)

            def tri_body(t, _):
                m0 = t * NSLOT
                g0 = front(0, m0, qq)
                g1 = front(1, m0 + 1, qq)
                back(0, m0, g0)
                g2 = front(2, m0 + 2, qq)
                back(1, m0 + 1, g1)
                back(2, m0 + 2, g2)
                return 0

            lax.fori_loop(0, n_tri, tri_body, 0)
            for x in range(NSLOT):
                drain_s(x)
                wait_e(x)
            plsc.subcore_barrier()
            pltpu.sync_copy(acc.at[pl.ds(s * zper, zper)],
                            out.at[qq, pl.ds(s * zper, zper), :])
            plsc.subcore_barrier()
            return 0

        lax.fori_loop(0, 2, quarter, 0)

    return spmm


def _make_gather(total):
    per_w = total // (NC * NS * 128)

    @functools.partial(
        pl.kernel,
        out_type=jax.ShapeDtypeStruct((total, D), jnp.float32),
        mesh=_mesh(),
        scratch_types=[
            pltpu.VMEM((per_w, 128), jnp.int32),
            pltpu.VMEM((2, 128, D), jnp.float32),
            pltpu.SemaphoreType.DMA,
            pltpu.SemaphoreType.DMA,
        ],
        compiler_params=pltpu.CompilerParams(use_tc_tiling_on_sc=False),
    )
    def gat(idx3d, x2, out, ibuf, gbuf, gsem, osem):
        c = lax.axis_index("c")
        s = lax.axis_index("s")
        w = s * NC + c
        pltpu.sync_copy(idx3d.at[w], ibuf)
        cps = [None] * per_w
        ocp = [None] * per_w
        for k in range(per_w):
            if k >= 2:
                ocp[k - 2].wait()
            cps[k] = pltpu.async_copy(x2.at[ibuf.at[k]], gbuf.at[k % 2], gsem)
            if k >= 1:
                cps[k - 1].wait()
                ocp[k - 1] = pltpu.async_copy(
                    gbuf.at[(k - 1) % 2],
                    out.at[pl.ds((w * per_w + k - 1) * 128, 128)], osem)
        kl = per_w - 1
        cps[kl].wait()
        ocp[kl] = pltpu.async_copy(
            gbuf.at[kl % 2], out.at[pl.ds((w * per_w + kl) * 128, 128)], osem)
        ocp[kl - 1].wait()
        ocp[kl].wait()

    return gat


def _transform_tc(x, a4, w, bias):
    n = x.shape[0]
    bn = 2000

    def body(x_ref, a0_ref, a1_ref, a2_ref, a3_ref, w_ref, b_ref, o_ref):
        wf = w_ref[...]
        acc = jnp.dot(x_ref[...], wf, preferred_element_type=jnp.float32)
        for qi, aref in enumerate((a0_ref, a1_ref, a2_ref, a3_ref)):
            acc += jnp.dot(aref[0], wf[qi * LANES:(qi + 1) * LANES, :],
                           preferred_element_type=jnp.float32)
        o_ref[...] = jnp.maximum(acc + b_ref[...], 0.0)

    qspec = pl.BlockSpec((1, bn, LANES),
                         lambda i, q=0: (0, i, 0))
    return pl.pallas_call(
        body,
        grid=(n // bn,),
        in_specs=[
            pl.BlockSpec((bn, D), lambda i: (i, 0)),
            pl.BlockSpec((1, bn, LANES), lambda i: (0, i, 0)),
            pl.BlockSpec((1, bn, LANES), lambda i: (1, i, 0)),
            pl.BlockSpec((1, bn, LANES), lambda i: (2, i, 0)),
            pl.BlockSpec((1, bn, LANES), lambda i: (3, i, 0)),
            pl.BlockSpec((D, D), lambda i: (0, 0)),
            pl.BlockSpec((1, D), lambda i: (0, 0)),
        ],
        out_specs=pl.BlockSpec((bn, D), lambda i: (i, 0)),
        out_shape=jax.ShapeDtypeStruct((n, D), jnp.float32),
    )(x, a4, a4, a4, a4, w, bias)


BSEQ = 256
NBLK = B // BSEQ


def _ln_in(x, g, b):
    m = x.mean(-1, keepdims=True)
    v = ((x - m) ** 2).mean(-1, keepdims=True)
    return (x - m) / jnp.sqrt(v + 1e-5) * g + b


def _seq_body(se_ref, oh_ref, tx_ref, len_ref, ug_ref, pos_ref, neg_ref,
              pb_ref, tW1_ref, tb1_ref, tW2_ref, tb2_ref, wih_ref, whh_ref,
              bih_ref, bhh_ref, lng_ref, lnb_ref, out_ref, xscr):
    lng = lng_ref[...]
    lnb = lnb_ref[...]
    tx = tx_ref[...]
    h1 = jnp.maximum(tx * tW1_ref[...][None] + tb1_ref[...][None], 0.0)
    te = jnp.dot(h1.reshape(L * BSEQ, D), tW2_ref[...],
                 preferred_element_type=jnp.float32) + tb2_ref[...]
    be = jnp.dot(oh_ref[...].reshape(L * BSEQ, NB), wih_ref[...][3, :NB, :],
                 preferred_element_type=jnp.float32)
    x = se_ref[...].reshape(L * BSEQ, D) + be + te
    xscr[...] = _ln_in(x, lng, lnb).reshape(L, BSEQ, D)

    lens = len_ref[...]

    def step(t, carry):
        h, res = carry
        xt = xscr[t]
        gr = (jnp.dot(xt, wih_ref[...][0], preferred_element_type=jnp.float32)
              + jnp.dot(h, whh_ref[...][0], preferred_element_type=jnp.float32)
              + bih_ref[...][0] + bhh_ref[...][0])
        gz = (jnp.dot(xt, wih_ref[...][1], preferred_element_type=jnp.float32)
              + jnp.dot(h, whh_ref[...][1], preferred_element_type=jnp.float32)
              + bih_ref[...][1] + bhh_ref[...][1])
        r = jax.nn.sigmoid(gr)
        z = jax.nn.sigmoid(gz)
        hn = (jnp.dot(h, whh_ref[...][2], preferred_element_type=jnp.float32)
              + bhh_ref[...][2])
        inn = (jnp.dot(xt, wih_ref[...][2], preferred_element_type=jnp.float32)
               + bih_ref[...][2])
        n = jnp.tanh(inn + r * hn)
        hnew = (1.0 - z) * n + z * h
        res = jnp.where(lens == t + 1, hnew, res)
        return hnew, res

    h0 = jnp.zeros((BSEQ, D), jnp.float32)
    _, res = lax.fori_loop(0, L, step, (h0, h0))

    uf = _ln_in(ug_ref[...] + res, lng, lnb)
    pos = pos_ref[...]
    neg = neg_ref[...]
    ps = jnp.sum(uf * pos, axis=-1, keepdims=True)
    ns = jnp.sum(uf * neg, axis=-1, keepdims=True)
    xm = ps - ns
    sp = jnp.maximum(-xm, 0.0) + jnp.log1p(jnp.exp(-jnp.abs(xm)))
    pb = pb_ref[...]
    bw = jnp.where(pb == 0, 1.0,
                   jnp.where(pb == 1, 1.25, jnp.where(pb == 2, 1.6, 2.1)))
    out_ref[0, 0, 0] = jnp.sum(sp * bw)
    out_ref[0, 0, 1] = jnp.sum(jnp.sqrt(jnp.sum(uf * uf, axis=-1)))
    out_ref[0, 0, 2] = jnp.sum(jnp.sqrt(jnp.sum(pos * pos, axis=-1)))
    out_ref[0, 0, 3] = jnp.sum(jnp.sqrt(jnp.sum(neg * neg, axis=-1)))


def _seq_tc(seT, ohT, txT, lens, ug, pos, neg, pb, t_W1, t_b1, t_W2, t_b2,
            wih4, whh3, bih3, bhh3, ln_g, ln_b):
    return pl.pallas_call(
        _seq_body,
        grid=(NBLK,),
        in_specs=[
            pl.BlockSpec((L, BSEQ, D), lambda i: (0, i, 0)),
            pl.BlockSpec((L, BSEQ, NB), lambda i: (0, i, 0)),
            pl.BlockSpec((L, BSEQ, 1), lambda i: (0, i, 0)),
            pl.BlockSpec((BSEQ, 1), lambda i: (i, 0)),
            pl.BlockSpec((BSEQ, D), lambda i: (i, 0)),
            pl.BlockSpec((BSEQ, D), lambda i: (i, 0)),
            pl.BlockSpec((BSEQ, D), lambda i: (i, 0)),
            pl.BlockSpec((BSEQ, 1), lambda i: (i, 0)),
            pl.BlockSpec((1, D), lambda i: (0, 0)),
            pl.BlockSpec((1, D), lambda i: (0, 0)),
            pl.BlockSpec((D, D), lambda i: (0, 0)),
            pl.BlockSpec((1, D), lambda i: (0, 0)),
            pl.BlockSpec((4, D, D), lambda i: (0, 0, 0)),
            pl.BlockSpec((3, D, D), lambda i: (0, 0, 0)),
            pl.BlockSpec((3, 1, D), lambda i: (0, 0, 0)),
            pl.BlockSpec((3, 1, D), lambda i: (0, 0, 0)),
            pl.BlockSpec((1, D), lambda i: (0, 0)),
            pl.BlockSpec((1, D), lambda i: (0, 0)),
        ],
        out_specs=pl.BlockSpec((1, 1, 4), lambda i: (i, 0, 0),
                               memory_space=pltpu.SMEM),
        out_shape=jax.ShapeDtypeStruct((NBLK, 1, 4), jnp.float32),
        scratch_shapes=[pltpu.VMEM((L, BSEQ, D), jnp.float32)],
    )(seT, ohT, txT, lens, ug, pos, neg, pb, t_W1, t_b1, t_W2, t_b2,
      wih4, whh3, bih3, bhh3, ln_g, ln_b)


def _pad_edges(rows, cols, vals):
    n = rows.shape[0]
    npad = (-n) % CHUNK
    if npad:
        rows = jnp.concatenate(
            [rows, NOUT + (jnp.arange(npad, dtype=jnp.int32) % LANES)])
        cols = jnp.concatenate([cols, jnp.zeros((npad,), jnp.int32)])
        vals = jnp.concatenate([vals, jnp.zeros((npad,), jnp.float32)])
    total = n + npad
    n_macro = total // CHUNK
    epk = jnp.stack([
        rows.reshape(total // 128, 128),
        (cols * 4).reshape(total // 128, 128),
        lax.bitcast_convert_type(vals, jnp.int32).reshape(total // 128, 128),
    ], axis=1)
    return epk, n_macro


def kernel(ui_rows, ui_cols, ui_vals, ii_rows, ii_cols, ii_vals, seq_items, seq_behaviors, seq_delta_days, seq_len, user_idx, pos_item_idx, neg_item_idx, pos_behavior, user_emb, item_emb, beh_emb, gnn_u_W, gnn_u_b, gnn_i_W, gnn_i_b, t_W1, t_b1, t_W2, t_b2, gru_Wih, gru_Whh, gru_bih, gru_bhh, ln_g, ln_b):
    i32 = jnp.int32
    ui_rows = ui_rows.astype(i32)
    ui_cols = ui_cols.astype(i32)
    ii_rows = ii_rows.astype(i32)
    ii_cols = ii_cols.astype(i32)

    i_rows = jnp.concatenate([ii_rows, ui_cols])
    i_cols = jnp.concatenate([ii_cols, ui_rows + I])
    i_vals = jnp.concatenate([ii_vals, ui_vals])

    epk_u, nmu = _pad_edges(ui_rows, ui_cols, ui_vals)
    epk_i, nmi = _pad_edges(i_rows, i_cols, i_vals)
    spmm_u = _make_spmm(nmu)
    spmm_i = _make_spmm(nmi)

    u, it = user_emb, item_emb
    for l in range(NL):
        x4 = jnp.concatenate([it, u], axis=0).reshape((I + U) * NQ, LANES)
        agg_u4 = spmm_u(epk_u, x4)
        agg_i4 = spmm_i(epk_i, x4)
        u = _transform_tc(u, agg_u4, gnn_u_W[l], gnn_u_b[l].reshape(1, D))
        it = _transform_tc(it, agg_i4, gnn_i_W[l], gnn_i_b[l].reshape(1, D))

    x2 = jnp.concatenate([it, u], axis=0)
    gidx = jnp.concatenate([
        seq_items.reshape(-1).astype(i32),
        pos_item_idx.astype(i32),
        neg_item_idx.astype(i32),
        user_idx.astype(i32) + I,
    ])
    total = gidx.shape[0]
    g = _make_gather(total)(
        gidx.reshape(NC * NS, total // (NC * NS * 128), 128), x2)
    se = g[:B * L].reshape(B, L, D)
    pos = g[B * L:B * L + B]
    neg = g[B * L + B:B * L + 2 * B]
    ug = g[B * L + 2 * B:]

    seT = jnp.swapaxes(se, 0, 1)
    ohT = jnp.swapaxes(
        jax.nn.one_hot(seq_behaviors, NB, dtype=jnp.float32), 0, 1)
    txT = jnp.swapaxes(jnp.log1p(seq_delta_days), 0, 1)[..., None]
    wih3 = jnp.stack(jnp.split(gru_Wih, 3, axis=1))
    behp = jnp.zeros((1, D, D), jnp.float32).at[0, :NB, :].set(0.35 * beh_emb)
    wih4 = jnp.concatenate([wih3, behp], axis=0)
    whh3 = jnp.stack(jnp.split(gru_Whh, 3, axis=1))
    bih3 = jnp.stack(jnp.split(gru_bih, 3)).reshape(3, 1, D)
    bhh3 = jnp.stack(jnp.split(gru_bhh, 3)).reshape(3, 1, D)

    partials = _seq_tc(
        seT, ohT, txT, seq_len.astype(i32).reshape(B, 1), ug, pos, neg,
        pos_behavior.astype(i32).reshape(B, 1),
        t_W1, t_b1.reshape(1, D), t_W2, t_b2.reshape(1, D),
        wih4, whh3, bih3, bhh3, ln_g.reshape(1, D), ln_b.reshape(1, D))
    sums = partials.reshape(NBLK, 4).sum(0)
    bpr = sums[0] / B
    reg = (sums[1] + sums[2] + sums[3]) / B * 1e-4
    return bpr + reg
